# baseline (speedup 1.0000x reference)
import jax
import jax.numpy as jnp
from jax.experimental import pallas as pl
from jax.experimental.pallas import tpu as pltpu

F32 = jnp.float32

_H, _W = 32, 32
_HO, _WO = 16, 16
_CIN, _COUT, _F = 4, 4, 8
_C0, _C1 = 128, 256
_NB = 2


def _zero_hbands(padh_ref, W):
    M, C = padh_ref.shape
    z = jnp.zeros((W, C), F32)
    padh_ref[0:W, :] = z
    padh_ref[M - W:M, :] = z


def _conv3x3(padh_ref, s_ref, x_flat, w_flat, bias, H, W, Cin):
    HW = H * W
    padh_ref[W:W + HW, :] = x_flat
    zc = jnp.zeros((H, 1, Cin), F32)
    for kh in range(3):
        band = padh_ref[kh * W:kh * W + HW, :].reshape(H, W, Cin)
        b0 = jnp.concatenate([zc, band[:, :W - 1, :]], axis=1)
        b2 = jnp.concatenate([band[:, 1:, :], zc], axis=1)
        k = 3 * kh
        s_ref[:, k * Cin:(k + 1) * Cin] = b0.reshape(HW, Cin)
        s_ref[:, (k + 1) * Cin:(k + 2) * Cin] = band.reshape(HW, Cin)
        s_ref[:, (k + 2) * Cin:(k + 3) * Cin] = b2.reshape(HW, Cin)
    return jnp.dot(s_ref[...], w_flat,
                   preferred_element_type=F32) + bias


def _mod_block(padh_ref, s_ref, x_flat, y_row,
               lw, lb, w1, b1, w2, b2, H, W, C):
    m = jnp.dot(y_row, lw, preferred_element_type=F32) + lb
    t = x_flat + m
    h = jnp.maximum(_conv3x3(padh_ref, s_ref, t, w1, b1, H, W, C), 0.0)
    r = _conv3x3(padh_ref, s_ref, h, w2, b2, H, W, C)
    return x_flat + r


def _unet_body(x_ref, y_ref,
               h0w_ref, h0b_ref, h1w_ref, h1b_ref,
               t1w_ref, t1b_ref, t0w_ref, t0b_ref,
               dec_ref, up_ref,
               d0lw_ref, d0lb_ref, d0w1_ref, d0b1_ref, d0w2_ref, d0b2_ref,
               d1lw_ref, d1lb_ref, d1w1_ref, d1b1_ref, d1w2_ref, d1b2_ref,
               a1lw_ref, a1lb_ref, a1w1_ref, a1b1_ref, a1w2_ref, a1b2_ref,
               a0lw_ref, a0lb_ref, a0w1_ref, a0b1_ref, a0w2_ref, a0b2_ref,
               o_ref,
               ph0, s0h, p0, s0, p1, s1, pt, st, cur0, skip0, cur1):
    _zero_hbands(ph0, _W)
    _zero_hbands(p0, _W)
    _zero_hbands(p1, _WO)
    _zero_hbands(pt, _W)

    y_row = y_ref[0]

    cur0[...] = _conv3x3(ph0, s0h, x_ref[0].reshape(_H * _W, _CIN),
                         h0w_ref[...], h0b_ref[...], _H, _W, _CIN)
    for b in range(_NB):
        cur0[...] = _mod_block(p0, s0, cur0[...], y_row,
                               d0lw_ref[b], d0lb_ref[b],
                               d0w1_ref[b], d0b1_ref[b],
                               d0w2_ref[b], d0b2_ref[b], _H, _W, _C0)
    skip0[...] = cur0[...]

    res = _conv3x3(p0, s0, cur0[...], h1w_ref[...], h1b_ref[...],
                   _H, _W, _C0)
    cur1[...] = jnp.dot(dec_ref[...], res,
                        preferred_element_type=F32)
    for b in range(_NB):
        cur1[...] = _mod_block(p1, s1, cur1[...], y_row,
                               d1lw_ref[b], d1lb_ref[b],
                               d1w1_ref[b], d1b1_ref[b],
                               d1w2_ref[b], d1b2_ref[b], _HO, _WO, _C1)

    for b in range(_NB):
        cur1[...] = _mod_block(p1, s1, cur1[...], y_row,
                               a1lw_ref[b], a1lb_ref[b],
                               a1w1_ref[b], a1b1_ref[b],
                               a1w2_ref[b], a1b2_ref[b], _HO, _WO, _C1)
    tin = jnp.dot(up_ref[...], cur1[...],
                  preferred_element_type=F32)
    res = _conv3x3(pt, st, tin, t1w_ref[...], t1b_ref[...],
                   _H, _W, _C1)
    cur0[...] = res + skip0[...]

    for b in range(_NB):
        cur0[...] = _mod_block(p0, s0, cur0[...], y_row,
                               a0lw_ref[b], a0lb_ref[b],
                               a0w1_ref[b], a0b1_ref[b],
                               a0w2_ref[b], a0b2_ref[b], _H, _W, _C0)
    res = _conv3x3(p0, s0, cur0[...], t0w_ref[...], t0b_ref[...],
                   _H, _W, _C0)
    o_ref[0] = res.reshape(_H, _W, _COUT)


def _decimation_matrix(H, W, Ho, Wo):
    rows = jnp.arange(Ho * Wo, dtype=jnp.int32)
    src = 2 * (rows // Wo) * W + 2 * (rows % Wo)
    return (jnp.arange(H * W, dtype=jnp.int32)[None, :] == src[:, None]
            ).astype(F32)


def _upsample_matrix(Hi, Wi, Ht, Wt):
    rows = jnp.arange(Ht * Wt, dtype=jnp.int32)
    src = ((rows // Wt) // 2) * Wi + (rows % Wt) // 2
    return (jnp.arange(Hi * Wi, dtype=jnp.int32)[None, :] == src[:, None]
            ).astype(F32)


def kernel(x, y, head_w_0, head_b_0, head_w_1, head_b_1, tail_w_0, tail_b_0, tail_w_1, tail_b_1, desc_0_0_lw, desc_0_0_lb, desc_0_0_w1, desc_0_0_b1, desc_0_0_w2, desc_0_0_b2, desc_0_1_lw, desc_0_1_lb, desc_0_1_w1, desc_0_1_b1, desc_0_1_w2, desc_0_1_b2, desc_1_0_lw, desc_1_0_lb, desc_1_0_w1, desc_1_0_b1, desc_1_0_w2, desc_1_0_b2, desc_1_1_lw, desc_1_1_lb, desc_1_1_w1, desc_1_1_b1, desc_1_1_w2, desc_1_1_b2, asc_0_0_lw, asc_0_0_lb, asc_0_0_w1, asc_0_0_b1, asc_0_0_w2, asc_0_0_b2, asc_0_1_lw, asc_0_1_lb, asc_0_1_w1, asc_0_1_b1, asc_0_1_w2, asc_0_1_b2, asc_1_0_lw, asc_1_0_lb, asc_1_0_w1, asc_1_0_b1, asc_1_0_w2, asc_1_0_b2, asc_1_1_lw, asc_1_1_lb, asc_1_1_w1, asc_1_1_b1, asc_1_1_w2, asc_1_1_b2):
    N = x.shape[0]

    x_nhwc = jnp.transpose(x, (0, 2, 3, 1)).astype(F32)
    y3 = y.astype(F32).reshape(N, 1, _F)

    cw = lambda w, k, c: w.reshape(9 * k, c).astype(F32)
    cb = lambda b: b.astype(F32).reshape(1, -1)
    stk_lw = lambda ws: jnp.stack([w.astype(F32) for w in ws])
    stk_w = lambda ws, k, c: jnp.stack([cw(w, k, c) for w in ws])
    stk_b = lambda bs: jnp.stack([cb(b) for b in bs])

    h0w, h0b = cw(head_w_0, _CIN, _C0), cb(head_b_0)
    h1w, h1b = cw(head_w_1, _C0, _C1), cb(head_b_1)
    t1w, t1b = cw(tail_w_1, _C1, _C0), cb(tail_b_1)
    t0w, t0b = cw(tail_w_0, _C0, _COUT), cb(tail_b_0)

    d0lw = stk_lw([desc_0_0_lw, desc_0_1_lw])
    d0lb = stk_b([desc_0_0_lb, desc_0_1_lb])
    d0w1 = stk_w([desc_0_0_w1, desc_0_1_w1], _C0, _C0)
    d0w2 = stk_w([desc_0_0_w2, desc_0_1_w2], _C0, _C0)
    d0b1 = stk_b([desc_0_0_b1, desc_0_1_b1])
    d0b2 = stk_b([desc_0_0_b2, desc_0_1_b2])
    d1lw = stk_lw([desc_1_0_lw, desc_1_1_lw])
    d1lb = stk_b([desc_1_0_lb, desc_1_1_lb])
    d1w1 = stk_w([desc_1_0_w1, desc_1_1_w1], _C1, _C1)
    d1w2 = stk_w([desc_1_0_w2, desc_1_1_w2], _C1, _C1)
    d1b1 = stk_b([desc_1_0_b1, desc_1_1_b1])
    d1b2 = stk_b([desc_1_0_b2, desc_1_1_b2])
    a1lw = stk_lw([asc_1_0_lw, asc_1_1_lw])
    a1lb = stk_b([asc_1_0_lb, asc_1_1_lb])
    a1w1 = stk_w([asc_1_0_w1, asc_1_1_w1], _C1, _C1)
    a1w2 = stk_w([asc_1_0_w2, asc_1_1_w2], _C1, _C1)
    a1b1 = stk_b([asc_1_0_b1, asc_1_1_b1])
    a1b2 = stk_b([asc_1_0_b2, asc_1_1_b2])
    a0lw = stk_lw([asc_0_0_lw, asc_0_1_lw])
    a0lb = stk_b([asc_0_0_lb, asc_0_1_lb])
    a0w1 = stk_w([asc_0_0_w1, asc_0_1_w1], _C0, _C0)
    a0w2 = stk_w([asc_0_0_w2, asc_0_1_w2], _C0, _C0)
    a0b1 = stk_b([asc_0_0_b1, asc_0_1_b1])
    a0b2 = stk_b([asc_0_0_b2, asc_0_1_b2])

    dec = _decimation_matrix(_H, _W, _HO, _WO)
    up = _upsample_matrix(_HO, _WO, _H, _W)

    wspec2 = lambda shape: pl.BlockSpec(shape, lambda n: (0, 0))
    wspec3 = lambda shape: pl.BlockSpec(shape, lambda n: (0, 0, 0))

    in_specs = [
        pl.BlockSpec((1, _H, _W, _CIN), lambda n: (n, 0, 0, 0)),
        pl.BlockSpec((1, 1, _F), lambda n: (n, 0, 0)),
        wspec2(h0w.shape), wspec2(h0b.shape),
        wspec2(h1w.shape), wspec2(h1b.shape),
        wspec2(t1w.shape), wspec2(t1b.shape),
        wspec2(t0w.shape), wspec2(t0b.shape),
        wspec2(dec.shape), wspec2(up.shape),
        wspec3(d0lw.shape), wspec3(d0lb.shape),
        wspec3(d0w1.shape), wspec3(d0b1.shape),
        wspec3(d0w2.shape), wspec3(d0b2.shape),
        wspec3(d1lw.shape), wspec3(d1lb.shape),
        wspec3(d1w1.shape), wspec3(d1b1.shape),
        wspec3(d1w2.shape), wspec3(d1b2.shape),
        wspec3(a1lw.shape), wspec3(a1lb.shape),
        wspec3(a1w1.shape), wspec3(a1b1.shape),
        wspec3(a1w2.shape), wspec3(a1b2.shape),
        wspec3(a0lw.shape), wspec3(a0lb.shape),
        wspec3(a0w1.shape), wspec3(a0b1.shape),
        wspec3(a0w2.shape), wspec3(a0b2.shape),
    ]
    inputs = [x_nhwc, y3, h0w, h0b, h1w, h1b, t1w, t1b, t0w, t0b, dec, up,
              d0lw, d0lb, d0w1, d0b1, d0w2, d0b2,
              d1lw, d1lb, d1w1, d1b1, d1w2, d1b2,
              a1lw, a1lb, a1w1, a1b1, a1w2, a1b2,
              a0lw, a0lb, a0w1, a0b1, a0w2, a0b2]

    HW, HOWO = _H * _W, _HO * _WO
    out = pl.pallas_call(
        _unet_body,
        out_shape=jax.ShapeDtypeStruct((N, _H, _W, _COUT), F32),
        grid=(N,),
        in_specs=in_specs,
        out_specs=pl.BlockSpec((1, _H, _W, _COUT), lambda n: (n, 0, 0, 0)),
        scratch_shapes=[
            pltpu.VMEM(((_H + 2) * _W, _CIN), F32),
            pltpu.VMEM((HW, 9 * _CIN), F32),
            pltpu.VMEM(((_H + 2) * _W, _C0), F32),
            pltpu.VMEM((HW, 9 * _C0), F32),
            pltpu.VMEM(((_HO + 2) * _WO, _C1), F32),
            pltpu.VMEM((HOWO, 9 * _C1), F32),
            pltpu.VMEM(((_H + 2) * _W, _C1), F32),
            pltpu.VMEM((HW, 9 * _C1), F32),
            pltpu.VMEM((HW, _C0), F32),
            pltpu.VMEM((HW, _C0), F32),
            pltpu.VMEM((HOWO, _C1), F32),
        ],
        compiler_params=pltpu.CompilerParams(
            dimension_semantics=("parallel",),
            vmem_limit_bytes=100 * 1024 * 1024),
    )(*inputs)

    return jnp.transpose(out, (0, 3, 1, 2))


# speedup vs baseline: 1.0190x; 1.0190x over previous
import jax
import jax.numpy as jnp
from jax.experimental import pallas as pl
from jax.experimental.pallas import tpu as pltpu

F32 = jnp.float32

_H, _W = 32, 32
_HO, _WO = 16, 16
_CIN, _COUT, _F = 4, 4, 8
_C0, _C1 = 128, 256
_NB = 2


def _zero_hbands(padh_ref, W):
    M, C = padh_ref.shape
    z = jnp.zeros((W, C), F32)
    padh_ref[0:W, :] = z
    padh_ref[M - W:M, :] = z


def _conv3x3(padh_ref, s_ref, x_flat, w_flat, bias, H, W, Cin):
    HW = H * W
    padh_ref[W:W + HW, :] = x_flat
    zc = jnp.zeros((H, 1, Cin), F32)
    for kh in range(3):
        band = padh_ref[kh * W:kh * W + HW, :].reshape(H, W, Cin)
        b0 = jnp.concatenate([zc, band[:, :W - 1, :]], axis=1)
        b2 = jnp.concatenate([band[:, 1:, :], zc], axis=1)
        k = 3 * kh
        s_ref[:, k * Cin:(k + 1) * Cin] = b0.reshape(HW, Cin)
        s_ref[:, (k + 1) * Cin:(k + 2) * Cin] = band.reshape(HW, Cin)
        s_ref[:, (k + 2) * Cin:(k + 3) * Cin] = b2.reshape(HW, Cin)
    return jnp.dot(s_ref[...], w_flat,
                   preferred_element_type=F32) + bias


def _mod_block(padh_ref, s_ref, x_flat, y_row,
               lw, lb, w1, b1, w2, b2, H, W, C):
    m = jnp.dot(y_row, lw, preferred_element_type=F32) + lb
    t = x_flat + m
    h = jnp.maximum(_conv3x3(padh_ref, s_ref, t, w1, b1, H, W, C), 0.0)
    r = _conv3x3(padh_ref, s_ref, h, w2, b2, H, W, C)
    return x_flat + r


def _unet_body(x_ref, y_ref,
               h0w_ref, h0b_ref, h1w_ref, h1b_ref,
               t1w_ref, t1b_ref, t0w_ref, t0b_ref,
               dec_ref, up_ref,
               d0lw_ref, d0lb_ref, d0w1_ref, d0b1_ref, d0w2_ref, d0b2_ref,
               d1lw_ref, d1lb_ref, d1w1_ref, d1b1_ref, d1w2_ref, d1b2_ref,
               a1lw_ref, a1lb_ref, a1w1_ref, a1b1_ref, a1w2_ref, a1b2_ref,
               a0lw_ref, a0lb_ref, a0w1_ref, a0b1_ref, a0w2_ref, a0b2_ref,
               o_ref,
               ph0a, s0ha, p0a, s0a, p1a, s1a, pta,
               ph0b, s0hb, p0b, s0b, p1b, s1b,
               st, cur0a, skip0a, cur1a, cur0b, skip0b, cur1b):
    for p in (ph0a, p0a, ph0b, p0b, pta):
        _zero_hbands(p, _W)
    _zero_hbands(p1a, _WO)
    _zero_hbands(p1b, _WO)

    ca = dict(i=0, ph0=ph0a, s0h=s0ha, p0=p0a, s0=s0a, p1=p1a, s1=s1a,
              pt=pta, st=st, cur0=cur0a, skip0=skip0a, cur1=cur1a)
    cb = dict(i=1, ph0=ph0b, s0h=s0hb, p0=p0b, s0=s0b, p1=p1b, s1=s1b,
              pt=pta, st=st, cur0=cur0b, skip0=skip0b, cur1=cur1b)
    for c in (ca, cb):
        c["y_row"] = y_ref[c["i"]]

    def head0(c):
        c["cur0"][...] = _conv3x3(
            c["ph0"], c["s0h"], x_ref[c["i"]].reshape(_H * _W, _CIN),
            h0w_ref[...], h0b_ref[...], _H, _W, _CIN)

    def blk0(c, lw, lb, w1, b1, w2, b2, j):
        c["cur0"][...] = _mod_block(c["p0"], c["s0"], c["cur0"][...],
                                    c["y_row"], lw[j], lb[j], w1[j], b1[j],
                                    w2[j], b2[j], _H, _W, _C0)

    def blk1(c, lw, lb, w1, b1, w2, b2, j):
        c["cur1"][...] = _mod_block(c["p1"], c["s1"], c["cur1"][...],
                                    c["y_row"], lw[j], lb[j], w1[j], b1[j],
                                    w2[j], b2[j], _HO, _WO, _C1)

    def head1(c):
        c["skip0"][...] = c["cur0"][...]
        res = _conv3x3(c["p0"], c["s0"], c["cur0"][...],
                       h1w_ref[...], h1b_ref[...], _H, _W, _C0)
        c["cur1"][...] = jnp.dot(dec_ref[...], res,
                                 preferred_element_type=F32)

    def tail1(c):
        tin = jnp.dot(up_ref[...], c["cur1"][...],
                      preferred_element_type=F32)
        res = _conv3x3(c["pt"], c["st"], tin,
                       t1w_ref[...], t1b_ref[...], _H, _W, _C1)
        c["cur0"][...] = res + c["skip0"][...]

    def tail0(c):
        res = _conv3x3(c["p0"], c["s0"], c["cur0"][...],
                       t0w_ref[...], t0b_ref[...], _H, _W, _C0)
        o_ref[c["i"]] = res.reshape(_H, _W, _COUT)

    stages = [head0]
    for j in range(_NB):
        stages.append(lambda c, j=j: blk0(
            c, d0lw_ref, d0lb_ref, d0w1_ref, d0b1_ref, d0w2_ref, d0b2_ref, j))
    stages.append(head1)
    for j in range(_NB):
        stages.append(lambda c, j=j: blk1(
            c, d1lw_ref, d1lb_ref, d1w1_ref, d1b1_ref, d1w2_ref, d1b2_ref, j))
    for j in range(_NB):
        stages.append(lambda c, j=j: blk1(
            c, a1lw_ref, a1lb_ref, a1w1_ref, a1b1_ref, a1w2_ref, a1b2_ref, j))
    stages.append(tail1)
    for j in range(_NB):
        stages.append(lambda c, j=j: blk0(
            c, a0lw_ref, a0lb_ref, a0w1_ref, a0b1_ref, a0w2_ref, a0b2_ref, j))
    stages.append(tail0)

    for stage in stages:
        stage(ca)
        stage(cb)


def _decimation_matrix(H, W, Ho, Wo):
    rows = jnp.arange(Ho * Wo, dtype=jnp.int32)
    src = 2 * (rows // Wo) * W + 2 * (rows % Wo)
    return (jnp.arange(H * W, dtype=jnp.int32)[None, :] == src[:, None]
            ).astype(F32)


def _upsample_matrix(Hi, Wi, Ht, Wt):
    rows = jnp.arange(Ht * Wt, dtype=jnp.int32)
    src = ((rows // Wt) // 2) * Wi + (rows % Wt) // 2
    return (jnp.arange(Hi * Wi, dtype=jnp.int32)[None, :] == src[:, None]
            ).astype(F32)


def kernel(x, y, head_w_0, head_b_0, head_w_1, head_b_1, tail_w_0, tail_b_0, tail_w_1, tail_b_1, desc_0_0_lw, desc_0_0_lb, desc_0_0_w1, desc_0_0_b1, desc_0_0_w2, desc_0_0_b2, desc_0_1_lw, desc_0_1_lb, desc_0_1_w1, desc_0_1_b1, desc_0_1_w2, desc_0_1_b2, desc_1_0_lw, desc_1_0_lb, desc_1_0_w1, desc_1_0_b1, desc_1_0_w2, desc_1_0_b2, desc_1_1_lw, desc_1_1_lb, desc_1_1_w1, desc_1_1_b1, desc_1_1_w2, desc_1_1_b2, asc_0_0_lw, asc_0_0_lb, asc_0_0_w1, asc_0_0_b1, asc_0_0_w2, asc_0_0_b2, asc_0_1_lw, asc_0_1_lb, asc_0_1_w1, asc_0_1_b1, asc_0_1_w2, asc_0_1_b2, asc_1_0_lw, asc_1_0_lb, asc_1_0_w1, asc_1_0_b1, asc_1_0_w2, asc_1_0_b2, asc_1_1_lw, asc_1_1_lb, asc_1_1_w1, asc_1_1_b1, asc_1_1_w2, asc_1_1_b2):
    N = x.shape[0]

    x_nhwc = jnp.transpose(x, (0, 2, 3, 1)).astype(F32)
    y3 = y.astype(F32).reshape(N, 1, _F)

    cw = lambda w, k, c: w.reshape(9 * k, c).astype(F32)
    cb = lambda b: b.astype(F32).reshape(1, -1)
    stk_lw = lambda ws: jnp.stack([w.astype(F32) for w in ws])
    stk_w = lambda ws, k, c: jnp.stack([cw(w, k, c) for w in ws])
    stk_b = lambda bs: jnp.stack([cb(b) for b in bs])

    h0w, h0b = cw(head_w_0, _CIN, _C0), cb(head_b_0)
    h1w, h1b = cw(head_w_1, _C0, _C1), cb(head_b_1)
    t1w, t1b = cw(tail_w_1, _C1, _C0), cb(tail_b_1)
    t0w, t0b = cw(tail_w_0, _C0, _COUT), cb(tail_b_0)

    d0lw = stk_lw([desc_0_0_lw, desc_0_1_lw])
    d0lb = stk_b([desc_0_0_lb, desc_0_1_lb])
    d0w1 = stk_w([desc_0_0_w1, desc_0_1_w1], _C0, _C0)
    d0w2 = stk_w([desc_0_0_w2, desc_0_1_w2], _C0, _C0)
    d0b1 = stk_b([desc_0_0_b1, desc_0_1_b1])
    d0b2 = stk_b([desc_0_0_b2, desc_0_1_b2])
    d1lw = stk_lw([desc_1_0_lw, desc_1_1_lw])
    d1lb = stk_b([desc_1_0_lb, desc_1_1_lb])
    d1w1 = stk_w([desc_1_0_w1, desc_1_1_w1], _C1, _C1)
    d1w2 = stk_w([desc_1_0_w2, desc_1_1_w2], _C1, _C1)
    d1b1 = stk_b([desc_1_0_b1, desc_1_1_b1])
    d1b2 = stk_b([desc_1_0_b2, desc_1_1_b2])
    a1lw = stk_lw([asc_1_0_lw, asc_1_1_lw])
    a1lb = stk_b([asc_1_0_lb, asc_1_1_lb])
    a1w1 = stk_w([asc_1_0_w1, asc_1_1_w1], _C1, _C1)
    a1w2 = stk_w([asc_1_0_w2, asc_1_1_w2], _C1, _C1)
    a1b1 = stk_b([asc_1_0_b1, asc_1_1_b1])
    a1b2 = stk_b([asc_1_0_b2, asc_1_1_b2])
    a0lw = stk_lw([asc_0_0_lw, asc_0_1_lw])
    a0lb = stk_b([asc_0_0_lb, asc_0_1_lb])
    a0w1 = stk_w([asc_0_0_w1, asc_0_1_w1], _C0, _C0)
    a0w2 = stk_w([asc_0_0_w2, asc_0_1_w2], _C0, _C0)
    a0b1 = stk_b([asc_0_0_b1, asc_0_1_b1])
    a0b2 = stk_b([asc_0_0_b2, asc_0_1_b2])

    dec = _decimation_matrix(_H, _W, _HO, _WO)
    up = _upsample_matrix(_HO, _WO, _H, _W)

    wspec2 = lambda shape: pl.BlockSpec(shape, lambda n: (0, 0))
    wspec3 = lambda shape: pl.BlockSpec(shape, lambda n: (0, 0, 0))

    in_specs = [
        pl.BlockSpec((2, _H, _W, _CIN), lambda n: (n, 0, 0, 0)),
        pl.BlockSpec((2, 1, _F), lambda n: (n, 0, 0)),
        wspec2(h0w.shape), wspec2(h0b.shape),
        wspec2(h1w.shape), wspec2(h1b.shape),
        wspec2(t1w.shape), wspec2(t1b.shape),
        wspec2(t0w.shape), wspec2(t0b.shape),
        wspec2(dec.shape), wspec2(up.shape),
        wspec3(d0lw.shape), wspec3(d0lb.shape),
        wspec3(d0w1.shape), wspec3(d0b1.shape),
        wspec3(d0w2.shape), wspec3(d0b2.shape),
        wspec3(d1lw.shape), wspec3(d1lb.shape),
        wspec3(d1w1.shape), wspec3(d1b1.shape),
        wspec3(d1w2.shape), wspec3(d1b2.shape),
        wspec3(a1lw.shape), wspec3(a1lb.shape),
        wspec3(a1w1.shape), wspec3(a1b1.shape),
        wspec3(a1w2.shape), wspec3(a1b2.shape),
        wspec3(a0lw.shape), wspec3(a0lb.shape),
        wspec3(a0w1.shape), wspec3(a0b1.shape),
        wspec3(a0w2.shape), wspec3(a0b2.shape),
    ]
    inputs = [x_nhwc, y3, h0w, h0b, h1w, h1b, t1w, t1b, t0w, t0b, dec, up,
              d0lw, d0lb, d0w1, d0b1, d0w2, d0b2,
              d1lw, d1lb, d1w1, d1b1, d1w2, d1b2,
              a1lw, a1lb, a1w1, a1b1, a1w2, a1b2,
              a0lw, a0lb, a0w1, a0b1, a0w2, a0b2]

    HW, HOWO = _H * _W, _HO * _WO
    per_sample = [
        pltpu.VMEM(((_H + 2) * _W, _CIN), F32),
        pltpu.VMEM((HW, 9 * _CIN), F32),
        pltpu.VMEM(((_H + 2) * _W, _C0), F32),
        pltpu.VMEM((HW, 9 * _C0), F32),
        pltpu.VMEM(((_HO + 2) * _WO, _C1), F32),
        pltpu.VMEM((HOWO, 9 * _C1), F32),
    ]
    out = pl.pallas_call(
        _unet_body,
        out_shape=jax.ShapeDtypeStruct((N, _H, _W, _COUT), F32),
        grid=(N // 2,),
        in_specs=in_specs,
        out_specs=pl.BlockSpec((2, _H, _W, _COUT), lambda n: (n, 0, 0, 0)),
        scratch_shapes=(
            per_sample + [pltpu.VMEM(((_H + 2) * _W, _C1), F32)]
            + per_sample
            + [pltpu.VMEM((HW, 9 * _C1), F32),
               pltpu.VMEM((HW, _C0), F32),
               pltpu.VMEM((HW, _C0), F32),
               pltpu.VMEM((HOWO, _C1), F32),
               pltpu.VMEM((HW, _C0), F32),
               pltpu.VMEM((HW, _C0), F32),
               pltpu.VMEM((HOWO, _C1), F32)]
        ),
        compiler_params=pltpu.CompilerParams(
            dimension_semantics=("parallel",),
            vmem_limit_bytes=100 * 1024 * 1024),
    )(*inputs)

    return jnp.transpose(out, (0, 3, 1, 2))
